# baseline (speedup 1.0000x reference)
import functools

import jax
import jax.numpy as jnp
from jax import lax
from jax.experimental import pallas as pl
from jax.experimental.pallas import tpu as pltpu
from jax.experimental.pallas import tpu_sc as plsc

SEQ = 200
EMBED_DIM = 64
BATCH = 4096
LANES = 16
NUM_CORES = 2
NUM_SUBCORES = 16
NUM_WORKERS = NUM_CORES * NUM_SUBCORES
B_TOTAL = BATCH * SEQ
ROWS_PER_W = B_TOTAL // NUM_WORKERS
CHUNK = SEQ
CHUNKS_PER_W = ROWS_PER_W // CHUNK
SCALE = 8.0
D_VECS = EMBED_DIM // LANES


def _sc_body(idx_hbm, table_hbm, pos_hbm, out_hbm, idx_v, rows_v, pos_v, sem):
    wid = lax.axis_index("s") * NUM_CORES + lax.axis_index("c")
    base = wid * ROWS_PER_W
    pltpu.sync_copy(pos_hbm, pos_v)

    @pl.loop(0, CHUNKS_PER_W)
    def _chunk(j):
        off = base + j * CHUNK
        pltpu.sync_copy(idx_hbm.at[pl.ds(off, CHUNK)], idx_v)
        pltpu.async_copy(table_hbm.at[idx_v], rows_v, sem).wait()

        @pl.loop(0, SEQ)
        def _row(s):
            for c in range(D_VECS):
                sl = pl.ds(c * LANES, LANES)
                rows_v[s, sl] = rows_v[s, sl] * SCALE + pos_v[s, sl]

        pltpu.sync_copy(rows_v, out_hbm.at[pl.ds(off, CHUNK)])


@jax.jit
def _embed(idx_flat, token_table, pos_table):
    grid_kernel = pl.kernel(
        _sc_body,
        out_type=jax.ShapeDtypeStruct((B_TOTAL, EMBED_DIM), jnp.float32),
        mesh=plsc.VectorSubcoreMesh(core_axis_name="c", subcore_axis_name="s"),
        scratch_types=[
            pltpu.VMEM((CHUNK,), jnp.int32),
            pltpu.VMEM((CHUNK, EMBED_DIM), jnp.float32),
            pltpu.VMEM((SEQ, EMBED_DIM), jnp.float32),
            pltpu.SemaphoreType.DMA,
        ],
        compiler_params=pltpu.CompilerParams(use_tc_tiling_on_sc=False),
    )
    return grid_kernel(idx_flat, token_table, pos_table)


def kernel(inputs, token_table, pos_table):
    idx_flat = inputs.reshape(-1).astype(jnp.int32)
    out = _embed(idx_flat, token_table, pos_table)
    return out.reshape(BATCH, SEQ, EMBED_DIM)
